# initial kernel scaffold (unmeasured)
import jax
import jax.numpy as jnp
from jax import lax
from jax.experimental import pallas as pl
from jax.experimental.pallas import tpu as pltpu

N_DEV = 8
B_PER = 2
SQ = 128
SKV = 128
D_MODEL = 512
HQ = 32
H_PER = 4
DH = 64
HD_PER = H_PER * DH


def kernel(x, Wq, K_ext, V_ext, Wo):
    def body(x_ref, wq_ref, k_hbm, v_hbm, wo_ref, out_ref,
             wq_all, wo_all, k_vmem, v_vmem,
             sq_sems, rq_sems, so_sems, ro_sems, kv_sems):
        my = lax.axis_index("i")
        right = jnp.mod(my + 1, N_DEV)
        left = jnp.mod(my - 1, N_DEV)

        k_copy = pltpu.make_async_copy(
            k_hbm.at[pl.ds(my * B_PER, B_PER)], k_vmem, kv_sems.at[0])
        v_copy = pltpu.make_async_copy(
            v_hbm.at[pl.ds(my * B_PER, B_PER)], v_vmem, kv_sems.at[1])
        k_copy.start()
        v_copy.start()

        wq_all[pl.ds(my, 1)] = wq_ref[...].astype(jnp.bfloat16)[None]
        wo_all[pl.ds(my, 1)] = wo_ref[...].astype(jnp.bfloat16)[None]

        barrier_sem = pltpu.get_barrier_semaphore()
        for nbr in (left, right):
            pl.semaphore_signal(barrier_sem, inc=1, device_id=(nbr,),
                                device_id_type=pl.DeviceIdType.MESH)
        pl.semaphore_wait(barrier_sem, 2)

        for h in range(N_DEV - 1):
            o = jnp.mod(my - h, N_DEV)
            rq = pltpu.make_async_remote_copy(
                src_ref=wq_all.at[o], dst_ref=wq_all.at[o],
                send_sem=sq_sems.at[h], recv_sem=rq_sems.at[h],
                device_id=(right,), device_id_type=pl.DeviceIdType.MESH)
            ro = pltpu.make_async_remote_copy(
                src_ref=wo_all.at[o], dst_ref=wo_all.at[o],
                send_sem=so_sems.at[h], recv_sem=ro_sems.at[h],
                device_id=(right,), device_id_type=pl.DeviceIdType.MESH)
            rq.start()
            ro.start()
            rq.wait()
            ro.wait()

        k_copy.wait()
        v_copy.wait()

        rows = B_PER * SQ
        x2d = x_ref[...].reshape(rows, D_MODEL).astype(jnp.bfloat16)

        ri = lax.broadcasted_iota(jnp.int32, (rows, rows), 0)
        ci = lax.broadcasted_iota(jnp.int32, (rows, rows), 1)
        qb = jnp.mod(ri, SQ) // 64
        kb = jnp.mod(ci, SKV) // 64
        same_batch = (ri // SQ) == (ci // SKV)
        mask = same_batch & ((qb == kb) | (jnp.mod(kb, 4) == jnp.mod(qb, 4)))

        acc = jnp.zeros((rows, D_MODEL), jnp.float32)
        for j in range(N_DEV):
            q = lax.dot_general(x2d, wq_all[j], (((1,), (0,)), ((), ())),
                                preferred_element_type=jnp.float32)
            ctx_cols = []
            for h in range(H_PER):
                gh = j * H_PER + h
                qh = q[:, h * DH:(h + 1) * DH].astype(jnp.bfloat16)
                kh = k_vmem[:, :, gh, :].reshape(rows, DH).astype(jnp.bfloat16)
                vh = v_vmem[:, :, gh, :].reshape(rows, DH).astype(jnp.bfloat16)
                s = lax.dot_general(qh, kh, (((1,), (1,)), ((), ())),
                                    preferred_element_type=jnp.float32) * 0.125
                s = jnp.where(mask, s, -1e9)
                m = jnp.max(s, axis=1, keepdims=True)
                e = jnp.exp(s - m)
                w = e / jnp.sum(e, axis=1, keepdims=True)
                ctx_cols.append(lax.dot_general(
                    w.astype(jnp.bfloat16), vh, (((1,), (0,)), ((), ())),
                    preferred_element_type=jnp.float32))
            ctx2d = jnp.concatenate(ctx_cols, axis=1).astype(jnp.bfloat16)
            acc = acc + lax.dot_general(
                ctx2d, wo_all[j], (((1,), (0,)), ((), ())),
                preferred_element_type=jnp.float32)

        out_ref[...] = acc.reshape(B_PER, SQ, D_MODEL)

    return pl.pallas_call(
        body,
        out_shape=jax.ShapeDtypeStruct((B_PER, SQ, D_MODEL), jnp.float32),
        in_specs=[
            pl.BlockSpec(memory_space=pltpu.VMEM),
            pl.BlockSpec(memory_space=pltpu.VMEM),
            pl.BlockSpec(memory_space=pltpu.ANY),
            pl.BlockSpec(memory_space=pltpu.ANY),
            pl.BlockSpec(memory_space=pltpu.VMEM),
        ],
        out_specs=pl.BlockSpec(memory_space=pltpu.VMEM),
        scratch_shapes=[
            pltpu.VMEM((N_DEV, D_MODEL, HD_PER), jnp.bfloat16),
            pltpu.VMEM((N_DEV, HD_PER, D_MODEL), jnp.bfloat16),
            pltpu.VMEM((B_PER, SKV, HQ, DH), jnp.float32),
            pltpu.VMEM((B_PER, SKV, HQ, DH), jnp.float32),
            pltpu.SemaphoreType.DMA((N_DEV - 1,)),
            pltpu.SemaphoreType.DMA((N_DEV - 1,)),
            pltpu.SemaphoreType.DMA((N_DEV - 1,)),
            pltpu.SemaphoreType.DMA((N_DEV - 1,)),
            pltpu.SemaphoreType.DMA((2,)),
        ],
        compiler_params=pltpu.CompilerParams(collective_id=0),
    )(x, Wq, K_ext, V_ext, Wo)


# baseline (device time: 129679 ns/iter reference)
import jax
import jax.numpy as jnp
from jax import lax
from jax.experimental import pallas as pl
from jax.experimental.pallas import tpu as pltpu

N_DEV = 8
B_PER = 2
SQ = 128
SKV = 128
D_MODEL = 512
HQ = 32
H_PER = 4
DH = 64
HD_PER = H_PER * DH


def kernel(x, Wq, K_ext, V_ext, Wo):
    def body(x_ref, wq_ref, k_hbm, v_hbm, wo_ref, out_ref,
             wq_all, wo_all, k_vmem, v_vmem,
             sq_sems, rq_sems, so_sems, ro_sems, kv_sems):
        my = lax.axis_index("i")
        right = jnp.mod(my + 1, N_DEV)
        left = jnp.mod(my - 1, N_DEV)

        k_copy = pltpu.make_async_copy(
            k_hbm.at[pl.ds(my * B_PER, B_PER)], k_vmem, kv_sems.at[0])
        v_copy = pltpu.make_async_copy(
            v_hbm.at[pl.ds(my * B_PER, B_PER)], v_vmem, kv_sems.at[1])
        k_copy.start()
        v_copy.start()

        wq_all[pl.ds(my, 1)] = wq_ref[...].astype(jnp.bfloat16)[None]
        wo_all[pl.ds(my, 1)] = wo_ref[...].astype(jnp.bfloat16)[None]

        barrier_sem = pltpu.get_barrier_semaphore()
        for nbr in (left, right):
            pl.semaphore_signal(barrier_sem, inc=1, device_id=(nbr,),
                                device_id_type=pl.DeviceIdType.MESH)
        pl.semaphore_wait(barrier_sem, 2)

        for h in range(N_DEV - 1):
            o = jnp.mod(my - h, N_DEV)
            rq = pltpu.make_async_remote_copy(
                src_ref=wq_all.at[o], dst_ref=wq_all.at[o],
                send_sem=sq_sems.at[h], recv_sem=rq_sems.at[h],
                device_id=(right,), device_id_type=pl.DeviceIdType.MESH)
            ro = pltpu.make_async_remote_copy(
                src_ref=wo_all.at[o], dst_ref=wo_all.at[o],
                send_sem=so_sems.at[h], recv_sem=ro_sems.at[h],
                device_id=(right,), device_id_type=pl.DeviceIdType.MESH)
            rq.start()
            ro.start()
            rq.wait()
            ro.wait()

        k_copy.wait()
        v_copy.wait()

        rows = B_PER * SQ
        x2d = x_ref[...].reshape(rows, D_MODEL).astype(jnp.bfloat16)

        ri = lax.broadcasted_iota(jnp.int32, (rows, rows), 0)
        ci = lax.broadcasted_iota(jnp.int32, (rows, rows), 1)
        qb = jnp.mod(ri, SQ) // 64
        kb = jnp.mod(ci, SKV) // 64
        same_batch = (ri // SQ) == (ci // SKV)
        mask = same_batch & ((qb == kb) | (jnp.mod(kb, 4) == jnp.mod(qb, 4)))

        acc = jnp.zeros((rows, D_MODEL), jnp.float32)
        for j in range(N_DEV):
            q = lax.dot_general(x2d, wq_all[j], (((1,), (0,)), ((), ())),
                                preferred_element_type=jnp.float32)
            ctx_cols = []
            for h in range(H_PER):
                gh = j * H_PER + h
                qh = q[:, h * DH:(h + 1) * DH].astype(jnp.bfloat16)
                kh = k_vmem[:, :, gh, :].reshape(rows, DH).astype(jnp.bfloat16)
                vh = v_vmem[:, :, gh, :].reshape(rows, DH).astype(jnp.bfloat16)
                s = lax.dot_general(qh, kh, (((1,), (1,)), ((), ())),
                                    preferred_element_type=jnp.float32) * 0.125
                s = jnp.where(mask, s, -1e9)
                m = jnp.max(s, axis=1, keepdims=True)
                e = jnp.exp(s - m)
                w = e / jnp.sum(e, axis=1, keepdims=True)
                ctx_cols.append(lax.dot_general(
                    w.astype(jnp.bfloat16), vh, (((1,), (0,)), ((), ())),
                    preferred_element_type=jnp.float32))
            ctx2d = jnp.concatenate(ctx_cols, axis=1).astype(jnp.bfloat16)
            acc = acc + lax.dot_general(
                ctx2d, wo_all[j], (((1,), (0,)), ((), ())),
                preferred_element_type=jnp.float32)

        out_ref[...] = acc.reshape(B_PER, SQ, D_MODEL)

    return pl.pallas_call(
        body,
        out_shape=jax.ShapeDtypeStruct((B_PER, SQ, D_MODEL), jnp.float32),
        in_specs=[
            pl.BlockSpec(memory_space=pltpu.VMEM),
            pl.BlockSpec(memory_space=pltpu.VMEM),
            pl.BlockSpec(memory_space=pltpu.MemorySpace.HBM),
            pl.BlockSpec(memory_space=pltpu.MemorySpace.HBM),
            pl.BlockSpec(memory_space=pltpu.VMEM),
        ],
        out_specs=pl.BlockSpec(memory_space=pltpu.VMEM),
        scratch_shapes=[
            pltpu.VMEM((N_DEV, D_MODEL, HD_PER), jnp.bfloat16),
            pltpu.VMEM((N_DEV, HD_PER, D_MODEL), jnp.bfloat16),
            pltpu.VMEM((B_PER, SKV, HQ, DH), jnp.float32),
            pltpu.VMEM((B_PER, SKV, HQ, DH), jnp.float32),
            pltpu.SemaphoreType.DMA((N_DEV - 1,)),
            pltpu.SemaphoreType.DMA((N_DEV - 1,)),
            pltpu.SemaphoreType.DMA((N_DEV - 1,)),
            pltpu.SemaphoreType.DMA((N_DEV - 1,)),
            pltpu.SemaphoreType.DMA((2,)),
        ],
        compiler_params=pltpu.CompilerParams(collective_id=0),
    )(x, Wq, K_ext, V_ext, Wo)


# device time: 108242 ns/iter; 1.1980x vs baseline; 1.1980x over previous
import jax
import jax.numpy as jnp
from jax import lax
from jax.experimental import pallas as pl
from jax.experimental.pallas import tpu as pltpu

N_DEV = 8
B_PER = 2
SQ = 128
SKV = 128
D_MODEL = 512
HQ = 32
H_PER = 4
DH = 64
HD_PER = H_PER * DH
ROWS = B_PER * SQ


def kernel(x, Wq, K_ext, V_ext, Wo):
    def body(x_ref, wq_ref, k_hbm, v_hbm, wo_ref, out_ref,
             wq_all, wo_all, k_heads, v_heads, k_bf, v_bf, ctx_all,
             send_sems, recv_sems, kv_sems):
        my = lax.axis_index("i")
        right = jnp.mod(my + 1, N_DEV)
        left = jnp.mod(my - 1, N_DEV)

        kv_copies = []
        for gh in range(HQ):
            kc = pltpu.make_async_copy(
                k_hbm.at[pl.ds(my * B_PER, B_PER), :, gh, :],
                k_heads.at[gh], kv_sems.at[0, gh])
            vc = pltpu.make_async_copy(
                v_hbm.at[pl.ds(my * B_PER, B_PER), :, gh, :],
                v_heads.at[gh], kv_sems.at[1, gh])
            kc.start()
            vc.start()
            kv_copies += [kc, vc]

        wq_own = wq_ref[...].reshape(D_MODEL, H_PER, DH).astype(jnp.bfloat16)
        wq_all[pl.ds(my, 1)] = jnp.transpose(wq_own, (1, 0, 2))[None]
        wo_all[pl.ds(my, 1)] = wo_ref[...].astype(jnp.bfloat16)[None]

        barrier_sem = pltpu.get_barrier_semaphore()
        for nbr in (left, right):
            pl.semaphore_signal(barrier_sem, inc=1, device_id=(nbr,),
                                device_id_type=pl.DeviceIdType.MESH)
        pl.semaphore_wait(barrier_sem, 2)

        def start_pair(o, dirn, hop, target):
            rq = pltpu.make_async_remote_copy(
                src_ref=wq_all.at[o], dst_ref=wq_all.at[o],
                send_sem=send_sems.at[dirn, 0, hop],
                recv_sem=recv_sems.at[dirn, 0, hop],
                device_id=(target,), device_id_type=pl.DeviceIdType.MESH)
            ro = pltpu.make_async_remote_copy(
                src_ref=wo_all.at[o], dst_ref=wo_all.at[o],
                send_sem=send_sems.at[dirn, 1, hop],
                recv_sem=recv_sems.at[dirn, 1, hop],
                device_id=(target,), device_id_type=pl.DeviceIdType.MESH)
            rq.start()
            ro.start()
            return rq, ro

        pairs = [start_pair(my, 0, 0, right), start_pair(my, 1, 0, left)]

        for c in kv_copies:
            c.wait()
        k_bf[...] = k_heads[...].reshape(HQ, ROWS, DH).astype(jnp.bfloat16)
        v_bf[...] = v_heads[...].reshape(HQ, ROWS, DH).astype(jnp.bfloat16)

        x2d = x_ref[...].reshape(ROWS, D_MODEL).astype(jnp.bfloat16)
        x_b = jnp.broadcast_to(x2d[None], (H_PER, ROWS, D_MODEL))

        ri = lax.broadcasted_iota(jnp.int32, (ROWS, ROWS), 0)
        ci = lax.broadcasted_iota(jnp.int32, (ROWS, ROWS), 1)
        qb = jnp.mod(ri, SQ) // 64
        kb = jnp.mod(ci, SKV) // 64
        same_batch = (ri // SQ) == (ci // SKV)
        mask = same_batch & ((qb == kb) | (jnp.mod(kb, 4) == jnp.mod(qb, 4)))
        maskb = mask[None]

        def compute_block(o):
            hs = o * H_PER
            wq_o = wq_all[pl.ds(o, 1)].reshape(H_PER, D_MODEL, DH)
            q = lax.dot_general(x_b, wq_o, (((2,), (1,)), ((0,), (0,))),
                                preferred_element_type=jnp.float32)
            qh = (q * 0.125).astype(jnp.bfloat16)
            k_o = k_bf[pl.ds(hs, H_PER)]
            v_o = v_bf[pl.ds(hs, H_PER)]
            s = lax.dot_general(qh, k_o, (((2,), (2,)), ((0,), (0,))),
                                preferred_element_type=jnp.float32)
            s = jnp.where(maskb, s, -1e9)
            m = jnp.max(s, axis=2, keepdims=True)
            e = jnp.exp(s - m)
            w = (e / jnp.sum(e, axis=2, keepdims=True)).astype(jnp.bfloat16)
            ctx = lax.dot_general(w, v_o, (((2,), (1,)), ((0,), (0,))),
                                  preferred_element_type=jnp.float32)
            ctx_all[pl.ds(hs, H_PER)] = ctx.astype(jnp.bfloat16)

        compute_block(my)

        o_r, o_l = my, my
        for h in range(N_DEV - 1):
            if h % 2 == 0:
                hr = h // 2
                pairs[2 * hr][0].wait_recv()
                pairs[2 * hr][1].wait_recv()
                o_r = jnp.mod(o_r - 1, N_DEV)
                if hr < 3:
                    pairs.append(start_pair(o_r, 0, hr + 1, right))
                compute_block(o_r)
            else:
                hl = h // 2
                pairs[2 * hl + 1][0].wait_recv()
                pairs[2 * hl + 1][1].wait_recv()
                o_l = jnp.mod(o_l + 1, N_DEV)
                if hl < 2:
                    pairs.append(start_pair(o_l, 1, hl + 1, left))
                compute_block(o_l)

        for rq, ro in pairs:
            rq.wait_send()
            ro.wait_send()

        ctx_t = jnp.transpose(ctx_all[...], (1, 0, 2)).reshape(ROWS, HQ * DH)
        wo_flat = wo_all[...].reshape(HQ * DH, D_MODEL)
        out2d = lax.dot_general(ctx_t, wo_flat, (((1,), (0,)), ((), ())),
                                preferred_element_type=jnp.float32)
        out_ref[...] = out2d.reshape(B_PER, SQ, D_MODEL)

    return pl.pallas_call(
        body,
        out_shape=jax.ShapeDtypeStruct((B_PER, SQ, D_MODEL), jnp.float32),
        in_specs=[
            pl.BlockSpec(memory_space=pltpu.VMEM),
            pl.BlockSpec(memory_space=pltpu.VMEM),
            pl.BlockSpec(memory_space=pltpu.MemorySpace.HBM),
            pl.BlockSpec(memory_space=pltpu.MemorySpace.HBM),
            pl.BlockSpec(memory_space=pltpu.VMEM),
        ],
        out_specs=pl.BlockSpec(memory_space=pltpu.VMEM),
        scratch_shapes=[
            pltpu.VMEM((N_DEV, H_PER, D_MODEL, DH), jnp.bfloat16),
            pltpu.VMEM((N_DEV, HD_PER, D_MODEL), jnp.bfloat16),
            pltpu.VMEM((HQ, B_PER, SKV, DH), jnp.float32),
            pltpu.VMEM((HQ, B_PER, SKV, DH), jnp.float32),
            pltpu.VMEM((HQ, ROWS, DH), jnp.bfloat16),
            pltpu.VMEM((HQ, ROWS, DH), jnp.bfloat16),
            pltpu.VMEM((HQ, ROWS, DH), jnp.bfloat16),
            pltpu.SemaphoreType.DMA((2, 2, 4)),
            pltpu.SemaphoreType.DMA((2, 2, 4)),
            pltpu.SemaphoreType.DMA((2, HQ)),
        ],
        compiler_params=pltpu.CompilerParams(collective_id=0),
    )(x, Wq, K_ext, V_ext, Wo)


# device time: 108068 ns/iter; 1.2000x vs baseline; 1.0016x over previous
import jax
import jax.numpy as jnp
from jax import lax
from jax.experimental import pallas as pl
from jax.experimental.pallas import tpu as pltpu

N_DEV = 8
B_PER = 2
SQ = 128
SKV = 128
D_MODEL = 512
HQ = 32
H_PER = 4
DH = 64
HD_PER = H_PER * DH
ROWS = B_PER * SQ
COMM = True
NB = H_PER * B_PER * 2


def kernel(x, Wq, K_ext, V_ext, Wo):
    def body(x_ref, wq_ref, k_hbm, v_hbm, wo_ref, out_ref,
             wq_all, wo_all, k_heads, v_heads, k_bf, v_bf, ctx_all,
             send_sems, recv_sems, kv_sems):
        my = lax.axis_index("i")
        right = jnp.mod(my + 1, N_DEV)
        left = jnp.mod(my - 1, N_DEV)

        kv_copies = []
        for gh in range(HQ):
            kc = pltpu.make_async_copy(
                k_hbm.at[pl.ds(my * B_PER, B_PER), :, gh, :],
                k_heads.at[gh], kv_sems.at[0, gh])
            vc = pltpu.make_async_copy(
                v_hbm.at[pl.ds(my * B_PER, B_PER), :, gh, :],
                v_heads.at[gh], kv_sems.at[1, gh])
            kc.start()
            vc.start()
            kv_copies += [kc, vc]

        wq_own = (wq_ref[...] * 0.125).reshape(D_MODEL, H_PER, DH).astype(jnp.bfloat16)
        wq_all[pl.ds(my, 1)] = jnp.transpose(wq_own, (1, 0, 2))[None]
        wo_all[pl.ds(my, 1)] = wo_ref[...].astype(jnp.bfloat16)[None]

        barrier_sem = pltpu.get_barrier_semaphore()
        for nbr in (left, right):
            pl.semaphore_signal(barrier_sem, inc=1, device_id=(nbr,),
                                device_id_type=pl.DeviceIdType.MESH)
        pl.semaphore_wait(barrier_sem, 2)

        def start_pair(o, dirn, hop, target):
            rq = pltpu.make_async_remote_copy(
                src_ref=wq_all.at[o], dst_ref=wq_all.at[o],
                send_sem=send_sems.at[dirn, 0, hop],
                recv_sem=recv_sems.at[dirn, 0, hop],
                device_id=(target,), device_id_type=pl.DeviceIdType.MESH)
            ro = pltpu.make_async_remote_copy(
                src_ref=wo_all.at[o], dst_ref=wo_all.at[o],
                send_sem=send_sems.at[dirn, 1, hop],
                recv_sem=recv_sems.at[dirn, 1, hop],
                device_id=(target,), device_id_type=pl.DeviceIdType.MESH)
            if COMM:
                rq.start()
                ro.start()
            return rq, ro

        pairs = [start_pair(my, 0, 0, right), start_pair(my, 1, 0, left)]

        for c in kv_copies:
            c.wait()
        k_bf[...] = k_heads[...].reshape(HQ, ROWS, DH).astype(jnp.bfloat16)
        v_bf[...] = v_heads[...].reshape(HQ, ROWS, DH).astype(jnp.bfloat16)

        x2d = x_ref[...].reshape(ROWS, D_MODEL).astype(jnp.bfloat16)
        x_b = jnp.broadcast_to(x2d[None], (H_PER, ROWS, D_MODEL))

        def compute_block(o):
            hs = o * H_PER
            wq_o = wq_all[pl.ds(o, 1)].reshape(H_PER, D_MODEL, DH)
            q = lax.dot_general(x_b, wq_o, (((2,), (1,)), ((0,), (0,))),
                                preferred_element_type=jnp.float32)
            qh = q.astype(jnp.bfloat16).reshape(NB, 64, DH)
            k_o = k_bf[pl.ds(hs, H_PER)].reshape(NB, 64, DH)
            v_o = v_bf[pl.ds(hs, H_PER)].reshape(NB, 64, DH)
            s = lax.dot_general(qh, k_o, (((2,), (2,)), ((0,), (0,))),
                                preferred_element_type=jnp.float32)
            e = jnp.exp(s)
            r = 1.0 / jnp.sum(e, axis=2, keepdims=True)
            w = (e * r).astype(jnp.bfloat16)
            ctx = lax.dot_general(w, v_o, (((2,), (1,)), ((0,), (0,))),
                                  preferred_element_type=jnp.float32)
            ctx_all[pl.ds(hs, H_PER)] = ctx.astype(jnp.bfloat16).reshape(
                H_PER, ROWS, DH)

        compute_block(my)

        o_r, o_l = my, my
        for h in range(N_DEV - 1):
            if h % 2 == 0:
                hr = h // 2
                if COMM:
                    pairs[2 * hr][0].wait_recv()
                    pairs[2 * hr][1].wait_recv()
                o_r = jnp.mod(o_r - 1, N_DEV)
                if hr < 3:
                    pairs.append(start_pair(o_r, 0, hr + 1, right))
                compute_block(o_r)
            else:
                hl = h // 2
                if COMM:
                    pairs[2 * hl + 1][0].wait_recv()
                    pairs[2 * hl + 1][1].wait_recv()
                o_l = jnp.mod(o_l + 1, N_DEV)
                if hl < 2:
                    pairs.append(start_pair(o_l, 1, hl + 1, left))
                compute_block(o_l)

        if COMM:
            for rq, ro in pairs:
                rq.wait_send()
                ro.wait_send()

        ctx_t = jnp.transpose(ctx_all[...], (1, 0, 2)).reshape(ROWS, HQ * DH)
        wo_flat = wo_all[...].reshape(HQ * DH, D_MODEL)
        out2d = lax.dot_general(ctx_t, wo_flat, (((1,), (0,)), ((), ())),
                                preferred_element_type=jnp.float32)
        out_ref[...] = out2d.reshape(B_PER, SQ, D_MODEL)

    return pl.pallas_call(
        body,
        out_shape=jax.ShapeDtypeStruct((B_PER, SQ, D_MODEL), jnp.float32),
        in_specs=[
            pl.BlockSpec(memory_space=pltpu.VMEM),
            pl.BlockSpec(memory_space=pltpu.VMEM),
            pl.BlockSpec(memory_space=pltpu.MemorySpace.HBM),
            pl.BlockSpec(memory_space=pltpu.MemorySpace.HBM),
            pl.BlockSpec(memory_space=pltpu.VMEM),
        ],
        out_specs=pl.BlockSpec(memory_space=pltpu.VMEM),
        scratch_shapes=[
            pltpu.VMEM((N_DEV, H_PER, D_MODEL, DH), jnp.bfloat16),
            pltpu.VMEM((N_DEV, HD_PER, D_MODEL), jnp.bfloat16),
            pltpu.VMEM((HQ, B_PER, SKV, DH), jnp.float32),
            pltpu.VMEM((HQ, B_PER, SKV, DH), jnp.float32),
            pltpu.VMEM((HQ, ROWS, DH), jnp.bfloat16),
            pltpu.VMEM((HQ, ROWS, DH), jnp.bfloat16),
            pltpu.VMEM((HQ, ROWS, DH), jnp.bfloat16),
            pltpu.SemaphoreType.DMA((2, 2, 4)),
            pltpu.SemaphoreType.DMA((2, 2, 4)),
            pltpu.SemaphoreType.DMA((2, HQ)),
        ],
        compiler_params=pltpu.CompilerParams(collective_id=0),
    )(x, Wq, K_ext, V_ext, Wo)


# device time: 68699 ns/iter; 1.8876x vs baseline; 1.5731x over previous
import jax
import jax.numpy as jnp
from jax import lax
from jax.experimental import pallas as pl
from jax.experimental.pallas import tpu as pltpu

N_DEV = 8
B_PER = 2
SQ = 128
SKV = 128
D_MODEL = 512
HQ = 32
H_PER = 4
DH = 64
HD_PER = H_PER * DH
ROWS = B_PER * SQ
NB = H_PER * B_PER * 2


def kernel(x, Wq, K_ext, V_ext, Wo):
    my_pos = lax.axis_index("i")
    K_my = lax.dynamic_slice_in_dim(K_ext, my_pos * B_PER, B_PER, axis=0)
    V_my = lax.dynamic_slice_in_dim(V_ext, my_pos * B_PER, B_PER, axis=0)

    def body(x_ref, wq_ref, k_in, v_in, wo_ref, out_ref,
             wq_all, wo_all, k_heads, v_heads, k_bf, v_bf, ctx_all,
             send_sems, recv_sems, kv_sems):
        my = lax.axis_index("i")
        right = jnp.mod(my + 1, N_DEV)
        left = jnp.mod(my - 1, N_DEV)

        kv_copies = []
        for gh in range(HQ):
            kc = pltpu.make_async_copy(
                k_in.at[:, :, gh, :], k_heads.at[gh], kv_sems.at[0, gh])
            vc = pltpu.make_async_copy(
                v_in.at[:, :, gh, :], v_heads.at[gh], kv_sems.at[1, gh])
            kc.start()
            vc.start()
            kv_copies += [kc, vc]

        wq_own = (wq_ref[...] * 0.125).reshape(
            D_MODEL, H_PER, DH).astype(jnp.bfloat16)
        wq_all[pl.ds(my, 1)] = jnp.transpose(wq_own, (1, 0, 2))[None]
        wo_all[pl.ds(my, 1)] = wo_ref[...].astype(jnp.bfloat16)[None]

        barrier_sem = pltpu.get_barrier_semaphore()
        for nbr in (left, right):
            pl.semaphore_signal(barrier_sem, inc=1, device_id=(nbr,),
                                device_id_type=pl.DeviceIdType.MESH)
        pl.semaphore_wait(barrier_sem, 2)

        def start_pair(o, dirn, hop, target):
            rq = pltpu.make_async_remote_copy(
                src_ref=wq_all.at[o], dst_ref=wq_all.at[o],
                send_sem=send_sems.at[dirn, 0, hop],
                recv_sem=recv_sems.at[dirn, 0, hop],
                device_id=(target,), device_id_type=pl.DeviceIdType.MESH)
            ro = pltpu.make_async_remote_copy(
                src_ref=wo_all.at[o], dst_ref=wo_all.at[o],
                send_sem=send_sems.at[dirn, 1, hop],
                recv_sem=recv_sems.at[dirn, 1, hop],
                device_id=(target,), device_id_type=pl.DeviceIdType.MESH)
            rq.start()
            ro.start()
            return rq, ro

        pairs = [start_pair(my, 0, 0, right), start_pair(my, 1, 0, left)]

        for c in kv_copies:
            c.wait()
        k_bf[...] = k_heads[...].reshape(HQ, ROWS, DH).astype(jnp.bfloat16)
        v_bf[...] = v_heads[...].reshape(HQ, ROWS, DH).astype(jnp.bfloat16)

        x2d = x_ref[...].reshape(ROWS, D_MODEL).astype(jnp.bfloat16)
        x_b = jnp.broadcast_to(x2d[None], (H_PER, ROWS, D_MODEL))

        def compute_block(o):
            hs = o * H_PER
            wq_o = wq_all[pl.ds(o, 1)].reshape(H_PER, D_MODEL, DH)
            q = lax.dot_general(x_b, wq_o, (((2,), (1,)), ((0,), (0,))),
                                preferred_element_type=jnp.float32)
            qh = q.astype(jnp.bfloat16).reshape(NB, 64, DH)
            k_o = k_bf[pl.ds(hs, H_PER)].reshape(NB, 64, DH)
            v_o = v_bf[pl.ds(hs, H_PER)].reshape(NB, 64, DH)
            s = lax.dot_general(qh, k_o, (((2,), (2,)), ((0,), (0,))),
                                preferred_element_type=jnp.float32)
            e = jnp.exp(s)
            r = 1.0 / jnp.sum(e, axis=2, keepdims=True)
            w = (e * r).astype(jnp.bfloat16)
            ctx = lax.dot_general(w, v_o, (((2,), (1,)), ((0,), (0,))),
                                  preferred_element_type=jnp.float32)
            ctx_all[pl.ds(hs, H_PER)] = ctx.astype(jnp.bfloat16).reshape(
                H_PER, ROWS, DH)

        compute_block(my)

        o_r, o_l = my, my
        for h in range(N_DEV - 1):
            if h % 2 == 0:
                hr = h // 2
                pairs[2 * hr][0].wait_recv()
                pairs[2 * hr][1].wait_recv()
                o_r = jnp.mod(o_r - 1, N_DEV)
                if hr < 3:
                    pairs.append(start_pair(o_r, 0, hr + 1, right))
                compute_block(o_r)
            else:
                hl = h // 2
                pairs[2 * hl + 1][0].wait_recv()
                pairs[2 * hl + 1][1].wait_recv()
                o_l = jnp.mod(o_l + 1, N_DEV)
                if hl < 2:
                    pairs.append(start_pair(o_l, 1, hl + 1, left))
                compute_block(o_l)

        for rq, ro in pairs:
            rq.wait_send()
            ro.wait_send()

        ctx_t = jnp.transpose(ctx_all[...], (1, 0, 2)).reshape(ROWS, HQ * DH)
        wo_flat = wo_all[...].reshape(HQ * DH, D_MODEL)
        out2d = lax.dot_general(ctx_t, wo_flat, (((1,), (0,)), ((), ())),
                                preferred_element_type=jnp.float32)
        out_ref[...] = out2d.reshape(B_PER, SQ, D_MODEL)

    return pl.pallas_call(
        body,
        out_shape=jax.ShapeDtypeStruct((B_PER, SQ, D_MODEL), jnp.float32),
        in_specs=[
            pl.BlockSpec(memory_space=pltpu.VMEM),
            pl.BlockSpec(memory_space=pltpu.VMEM),
            pl.BlockSpec(memory_space=pltpu.VMEM),
            pl.BlockSpec(memory_space=pltpu.VMEM),
            pl.BlockSpec(memory_space=pltpu.VMEM),
        ],
        out_specs=pl.BlockSpec(memory_space=pltpu.VMEM),
        scratch_shapes=[
            pltpu.VMEM((N_DEV, H_PER, D_MODEL, DH), jnp.bfloat16),
            pltpu.VMEM((N_DEV, HD_PER, D_MODEL), jnp.bfloat16),
            pltpu.VMEM((HQ, B_PER, SKV, DH), jnp.float32),
            pltpu.VMEM((HQ, B_PER, SKV, DH), jnp.float32),
            pltpu.VMEM((HQ, ROWS, DH), jnp.bfloat16),
            pltpu.VMEM((HQ, ROWS, DH), jnp.bfloat16),
            pltpu.VMEM((HQ, ROWS, DH), jnp.bfloat16),
            pltpu.SemaphoreType.DMA((2, 2, 4)),
            pltpu.SemaphoreType.DMA((2, 2, 4)),
            pltpu.SemaphoreType.DMA((2, HQ)),
        ],
        compiler_params=pltpu.CompilerParams(collective_id=0),
    )(x, Wq, K_my, V_my, Wo)


# device time: 68426 ns/iter; 1.8952x vs baseline; 1.0040x over previous
import jax
import jax.numpy as jnp
from jax import lax
from jax.experimental import pallas as pl
from jax.experimental.pallas import tpu as pltpu

N_DEV = 8
B_PER = 2
SQ = 128
SKV = 128
D_MODEL = 512
HQ = 32
H_PER = 4
DH = 64
HD_PER = H_PER * DH
ROWS = B_PER * SQ
NB = H_PER * B_PER * 2


def kernel(x, Wq, K_ext, V_ext, Wo):
    my_pos = lax.axis_index("i")
    K_my = lax.dynamic_slice_in_dim(K_ext, my_pos * B_PER, B_PER, axis=0)
    V_my = lax.dynamic_slice_in_dim(V_ext, my_pos * B_PER, B_PER, axis=0)

    def body(x_ref, wq_ref, k_in, v_in, wo_ref, out_ref,
             wq_all, wo_all, k_heads, v_heads, k_bf, v_bf, ctx_all,
             send_sems, recv_sems, kv_sems):
        my = lax.axis_index("i")
        right = jnp.mod(my + 1, N_DEV)
        left = jnp.mod(my - 1, N_DEV)
        zp = jnp.mod(my + 4, N_DEV)

        kv_copies = []
        for gh in range(HQ):
            kc = pltpu.make_async_copy(
                k_in.at[:, :, gh, :], k_heads.at[gh], kv_sems.at[0, gh])
            vc = pltpu.make_async_copy(
                v_in.at[:, :, gh, :], v_heads.at[gh], kv_sems.at[1, gh])
            kc.start()
            vc.start()
            kv_copies += [kc, vc]

        wq_own = (wq_ref[...] * 0.125).reshape(
            D_MODEL, H_PER, DH).astype(jnp.bfloat16)
        wq_all[pl.ds(my, 1)] = jnp.transpose(wq_own, (1, 0, 2))[None]
        wo_all[pl.ds(my, 1)] = wo_ref[...].astype(jnp.bfloat16)[None]

        barrier_sem = pltpu.get_barrier_semaphore()
        for nbr in (left, right, zp):
            pl.semaphore_signal(barrier_sem, inc=1, device_id=(nbr,),
                                device_id_type=pl.DeviceIdType.MESH)
        pl.semaphore_wait(barrier_sem, 3)

        def start_pair(o, dirn, hop, target):
            rq = pltpu.make_async_remote_copy(
                src_ref=wq_all.at[o], dst_ref=wq_all.at[o],
                send_sem=send_sems.at[dirn, 0, hop],
                recv_sem=recv_sems.at[dirn, 0, hop],
                device_id=(target,), device_id_type=pl.DeviceIdType.MESH)
            ro = pltpu.make_async_remote_copy(
                src_ref=wo_all.at[o], dst_ref=wo_all.at[o],
                send_sem=send_sems.at[dirn, 1, hop],
                recv_sem=recv_sems.at[dirn, 1, hop],
                device_id=(target,), device_id_type=pl.DeviceIdType.MESH)
            rq.start()
            ro.start()
            return rq, ro

        pairs = [start_pair(my, 0, 0, right), start_pair(my, 1, 0, left)]
        pair_z = start_pair(my, 2, 0, zp)

        for c in kv_copies:
            c.wait()
        k_bf[...] = k_heads[...].reshape(HQ, ROWS, DH).astype(jnp.bfloat16)
        v_bf[...] = v_heads[...].reshape(HQ, ROWS, DH).astype(jnp.bfloat16)

        x2d = x_ref[...].reshape(ROWS, D_MODEL).astype(jnp.bfloat16)
        x_b = jnp.broadcast_to(x2d[None], (H_PER, ROWS, D_MODEL))

        def compute_block(o):
            hs = o * H_PER
            wq_o = wq_all[pl.ds(o, 1)].reshape(H_PER, D_MODEL, DH)
            q = lax.dot_general(x_b, wq_o, (((2,), (1,)), ((0,), (0,))),
                                preferred_element_type=jnp.float32)
            qh = q.astype(jnp.bfloat16).reshape(NB, 64, DH)
            k_o = k_bf[pl.ds(hs, H_PER)].reshape(NB, 64, DH)
            v_o = v_bf[pl.ds(hs, H_PER)].reshape(NB, 64, DH)
            s = lax.dot_general(qh, k_o, (((2,), (2,)), ((0,), (0,))),
                                preferred_element_type=jnp.float32)
            e = jnp.exp(s)
            r = 1.0 / jnp.sum(e, axis=2, keepdims=True)
            w = (e * r).astype(jnp.bfloat16)
            ctx = lax.dot_general(w, v_o, (((2,), (1,)), ((0,), (0,))),
                                  preferred_element_type=jnp.float32)
            ctx_all[pl.ds(hs, H_PER)] = ctx.astype(jnp.bfloat16).reshape(
                H_PER, ROWS, DH)

        compute_block(my)

        pair_z[0].wait_recv()
        pair_z[1].wait_recv()
        compute_block(zp)

        o_r, o_l = my, my
        for h in range(3):
            pairs[2 * h][0].wait_recv()
            pairs[2 * h][1].wait_recv()
            o_r = jnp.mod(o_r - 1, N_DEV)
            if h < 2:
                pairs.append(start_pair(o_r, 0, h + 1, right))
            compute_block(o_r)

            pairs[2 * h + 1][0].wait_recv()
            pairs[2 * h + 1][1].wait_recv()
            o_l = jnp.mod(o_l + 1, N_DEV)
            if h < 2:
                pairs.append(start_pair(o_l, 1, h + 1, left))
            compute_block(o_l)

        for rq, ro in pairs + [pair_z]:
            rq.wait_send()
            ro.wait_send()

        ctx_t = jnp.transpose(ctx_all[...], (1, 0, 2)).reshape(ROWS, HQ * DH)
        wo_flat = wo_all[...].reshape(HQ * DH, D_MODEL)
        out2d = lax.dot_general(ctx_t, wo_flat, (((1,), (0,)), ((), ())),
                                preferred_element_type=jnp.float32)
        out_ref[...] = out2d.reshape(B_PER, SQ, D_MODEL)

    return pl.pallas_call(
        body,
        out_shape=jax.ShapeDtypeStruct((B_PER, SQ, D_MODEL), jnp.float32),
        in_specs=[
            pl.BlockSpec(memory_space=pltpu.VMEM),
            pl.BlockSpec(memory_space=pltpu.VMEM),
            pl.BlockSpec(memory_space=pltpu.VMEM),
            pl.BlockSpec(memory_space=pltpu.VMEM),
            pl.BlockSpec(memory_space=pltpu.VMEM),
        ],
        out_specs=pl.BlockSpec(memory_space=pltpu.VMEM),
        scratch_shapes=[
            pltpu.VMEM((N_DEV, H_PER, D_MODEL, DH), jnp.bfloat16),
            pltpu.VMEM((N_DEV, HD_PER, D_MODEL), jnp.bfloat16),
            pltpu.VMEM((HQ, B_PER, SKV, DH), jnp.float32),
            pltpu.VMEM((HQ, B_PER, SKV, DH), jnp.float32),
            pltpu.VMEM((HQ, ROWS, DH), jnp.bfloat16),
            pltpu.VMEM((HQ, ROWS, DH), jnp.bfloat16),
            pltpu.VMEM((HQ, ROWS, DH), jnp.bfloat16),
            pltpu.SemaphoreType.DMA((3, 2, 4)),
            pltpu.SemaphoreType.DMA((3, 2, 4)),
            pltpu.SemaphoreType.DMA((2, HQ)),
        ],
        compiler_params=pltpu.CompilerParams(collective_id=0),
    )(x, Wq, K_my, V_my, Wo)


# device time: 56630 ns/iter; 2.2899x vs baseline; 1.2083x over previous
import jax
import jax.numpy as jnp
from jax import lax
from jax.experimental import pallas as pl
from jax.experimental.pallas import tpu as pltpu

N_DEV = 8
B_PER = 2
SQ = 128
SKV = 128
D_MODEL = 512
HQ = 32
H_PER = 4
DH = 64
HD_PER = H_PER * DH
ROWS = B_PER * SQ
NB = H_PER * B_PER * 2


def kernel(x, Wq, K_ext, V_ext, Wo):
    my_pos = lax.axis_index("i")
    K_my = jnp.transpose(lax.dynamic_slice_in_dim(
        K_ext, my_pos * B_PER, B_PER, axis=0).astype(jnp.bfloat16),
        (2, 0, 1, 3))
    V_my = jnp.transpose(lax.dynamic_slice_in_dim(
        V_ext, my_pos * B_PER, B_PER, axis=0).astype(jnp.bfloat16),
        (2, 0, 1, 3))

    def body(x_ref, wq_ref, k_in, v_in, wo_ref, out_ref,
             wq_all, wo_all, ctx_all, send_sems, recv_sems):
        my = lax.axis_index("i")
        right = jnp.mod(my + 1, N_DEV)
        left = jnp.mod(my - 1, N_DEV)
        zp = jnp.mod(my + 4, N_DEV)

        wq_own = (wq_ref[...] * 0.125).reshape(
            D_MODEL, H_PER, DH).astype(jnp.bfloat16)
        wq_all[pl.ds(my, 1)] = jnp.transpose(wq_own, (1, 0, 2))[None]
        wo_all[pl.ds(my, 1)] = wo_ref[...].astype(jnp.bfloat16)[None]

        barrier_sem = pltpu.get_barrier_semaphore()
        for nbr in (left, right, zp):
            pl.semaphore_signal(barrier_sem, inc=1, device_id=(nbr,),
                                device_id_type=pl.DeviceIdType.MESH)
        pl.semaphore_wait(barrier_sem, 3)

        def start_pair(o, dirn, hop, target):
            rq = pltpu.make_async_remote_copy(
                src_ref=wq_all.at[o], dst_ref=wq_all.at[o],
                send_sem=send_sems.at[dirn, 0, hop],
                recv_sem=recv_sems.at[dirn, 0, hop],
                device_id=(target,), device_id_type=pl.DeviceIdType.MESH)
            ro = pltpu.make_async_remote_copy(
                src_ref=wo_all.at[o], dst_ref=wo_all.at[o],
                send_sem=send_sems.at[dirn, 1, hop],
                recv_sem=recv_sems.at[dirn, 1, hop],
                device_id=(target,), device_id_type=pl.DeviceIdType.MESH)
            rq.start()
            ro.start()
            return rq, ro

        pairs = [start_pair(my, 0, 0, right), start_pair(my, 1, 0, left)]
        pair_z = start_pair(my, 2, 0, zp)


        x2d = x_ref[...].reshape(ROWS, D_MODEL).astype(jnp.bfloat16)
        x_b = jnp.broadcast_to(x2d[None], (H_PER, ROWS, D_MODEL))

        def compute_block(o):
            hs = o * H_PER
            wq_o = wq_all[pl.ds(o, 1)].reshape(H_PER, D_MODEL, DH)
            q = lax.dot_general(x_b, wq_o, (((2,), (1,)), ((0,), (0,))),
                                preferred_element_type=jnp.float32)
            qh = q.astype(jnp.bfloat16).reshape(NB, 64, DH)
            k_o = k_in[pl.ds(hs, H_PER)].reshape(NB, 64, DH)
            v_o = v_in[pl.ds(hs, H_PER)].reshape(NB, 64, DH)
            s = lax.dot_general(qh, k_o, (((2,), (2,)), ((0,), (0,))),
                                preferred_element_type=jnp.float32)
            e = jnp.exp(s)
            r = 1.0 / jnp.sum(e, axis=2, keepdims=True)
            w = (e * r).astype(jnp.bfloat16)
            ctx = lax.dot_general(w, v_o, (((2,), (1,)), ((0,), (0,))),
                                  preferred_element_type=jnp.float32)
            ctx_all[pl.ds(hs, H_PER)] = ctx.astype(jnp.bfloat16).reshape(
                H_PER, ROWS, DH)

        compute_block(my)

        pair_z[0].wait_recv()
        pair_z[1].wait_recv()
        compute_block(zp)

        o_r, o_l = my, my
        for h in range(3):
            pairs[2 * h][0].wait_recv()
            pairs[2 * h][1].wait_recv()
            o_r = jnp.mod(o_r - 1, N_DEV)
            if h < 2:
                pairs.append(start_pair(o_r, 0, h + 1, right))
            compute_block(o_r)

            pairs[2 * h + 1][0].wait_recv()
            pairs[2 * h + 1][1].wait_recv()
            o_l = jnp.mod(o_l + 1, N_DEV)
            if h < 2:
                pairs.append(start_pair(o_l, 1, h + 1, left))
            compute_block(o_l)

        for rq, ro in pairs + [pair_z]:
            rq.wait_send()
            ro.wait_send()

        ctx_t = jnp.transpose(ctx_all[...], (1, 0, 2)).reshape(ROWS, HQ * DH)
        wo_flat = wo_all[...].reshape(HQ * DH, D_MODEL)
        out2d = lax.dot_general(ctx_t, wo_flat, (((1,), (0,)), ((), ())),
                                preferred_element_type=jnp.float32)
        out_ref[...] = out2d.reshape(B_PER, SQ, D_MODEL)

    return pl.pallas_call(
        body,
        out_shape=jax.ShapeDtypeStruct((B_PER, SQ, D_MODEL), jnp.float32),
        in_specs=[
            pl.BlockSpec(memory_space=pltpu.VMEM),
            pl.BlockSpec(memory_space=pltpu.VMEM),
            pl.BlockSpec(memory_space=pltpu.VMEM),
            pl.BlockSpec(memory_space=pltpu.VMEM),
            pl.BlockSpec(memory_space=pltpu.VMEM),
        ],
        out_specs=pl.BlockSpec(memory_space=pltpu.VMEM),
        scratch_shapes=[
            pltpu.VMEM((N_DEV, H_PER, D_MODEL, DH), jnp.bfloat16),
            pltpu.VMEM((N_DEV, HD_PER, D_MODEL), jnp.bfloat16),
            pltpu.VMEM((HQ, ROWS, DH), jnp.bfloat16),
            pltpu.SemaphoreType.DMA((3, 2, 4)),
            pltpu.SemaphoreType.DMA((3, 2, 4)),
        ],
        compiler_params=pltpu.CompilerParams(collective_id=0),
    )(x, Wq, K_my, V_my, Wo)
